# baseline (speedup 1.0000x reference)
import functools

import jax
import jax.numpy as jnp
from jax import lax
from jax.experimental import pallas as pl
from jax.experimental.pallas import tpu as pltpu
from jax.experimental.pallas import tpu_sc as plsc

B = 16384
EMB = 64
ITEM_V = 1000000
PAIR_H = 499968
PAIR_R = 500032
PAIR_SPLIT = 500032
NFULL = 3906
NC = 2
NS = 16
NW = NC * NS
BPW = B // NW
CH = 128
NCH = BPW // CH
IR = B // CH
NBND = 1024


def _sc_transpose_body(tab, tail, out, bl0, br0, bl1, br1, tl, ob0, ob1,
                       si0, si1, so0, so1):
    wid = lax.axis_index("s") * NC + lax.axis_index("c")
    rows16 = lax.iota(jnp.int32, 16)
    nblk = jnp.where(wid < NFULL - (NFULL // NW) * NW, NFULL // NW + 1,
                     NFULL // NW)
    bls = [bl0, bl1]
    brs = [br0, br1]
    obs = [ob0, ob1]
    sis = [si0, si1]
    sos = [so0, so1]

    def fire_in(t, b):
        c0 = pl.multiple_of((wid + t * NW) * 128, 128)
        pltpu.async_copy(tab.at[:, pl.ds(c0, 128)], bls[b], sis[b])
        pltpu.async_copy(tab.at[:, pl.ds(c0 + PAIR_H, 128)], brs[b], sis[b])

    def transpose_block(bl, br, ob):
        def row(r, _):
            for g in range(8):
                src = bl if g < 4 else br
                d0 = (g % 4) * 16
                v = plsc.load_gather(
                    src, [d0 + rows16, jnp.zeros((16,), jnp.int32) + r])
                ob[r, pl.ds(g * 16, 16)] = v
            return 0
        lax.fori_loop(0, 128, row, 0)

    fire_in(0, 0)

    def step(t, _):
        for b in range(2):
            tt = 2 * t + b

            @pl.when(tt < nblk)
            def _():
                @pl.when(tt + 1 < nblk)
                def _():
                    fire_in(tt + 1, 1 - b)
                pltpu.make_async_copy(tab.at[:, pl.ds(0, 128)], bls[b],
                                      sis[b]).wait()
                pltpu.make_async_copy(tab.at[:, pl.ds(0, 128)], brs[b],
                                      sis[b]).wait()

                @pl.when(tt >= 2)
                def _():
                    pltpu.make_async_copy(obs[b], out.at[pl.ds(0, 128)],
                                          sos[b]).wait()
                transpose_block(bls[b], brs[b], obs[b])
                c0 = pl.multiple_of((wid + tt * NW) * 128, 128)
                pltpu.async_copy(obs[b], out.at[pl.ds(c0, 128)], sos[b])
        return 0

    lax.fori_loop(0, (NFULL // NW + 2) // 2, step, 0)
    for b in range(2):
        pltpu.make_async_copy(obs[b], out.at[pl.ds(0, 128)], sos[b]).wait()

    @pl.when(wid == NFULL % NW)
    def _():
        c0 = NFULL * 128
        pltpu.async_copy(tab.at[:, pl.ds(c0, 128)], bl0, si0).wait()
        pltpu.sync_copy(tail, tl)

        def row(r, _):
            for g in range(8):
                src = bl0 if g < 4 else tl
                d0 = (g % 4) * 16
                v = plsc.load_gather(
                    src, [d0 + rows16, jnp.zeros((16,), jnp.int32) + r])
                ob0[r, pl.ds(g * 16, 16)] = v
            return 0
        lax.fori_loop(0, PAIR_R - NFULL * 128, row, 0)
        pltpu.sync_copy(ob0.at[pl.ds(0, PAIR_R - NFULL * 128)],
                        out.at[pl.ds(c0, PAIR_R - NFULL * 128)])


def _sc_transpose(item_tt, tail):
    f = functools.partial(
        pl.kernel,
        out_type=jax.ShapeDtypeStruct((PAIR_R, 2 * EMB), jnp.float32),
        mesh=plsc.VectorSubcoreMesh(core_axis_name="c", subcore_axis_name="s"),
        scratch_types=[
            pltpu.VMEM((EMB, 128), jnp.float32),
            pltpu.VMEM((EMB, 128), jnp.float32),
            pltpu.VMEM((EMB, 128), jnp.float32),
            pltpu.VMEM((EMB, 128), jnp.float32),
            pltpu.VMEM((EMB, EMB), jnp.float32),
            pltpu.VMEM((128, 128), jnp.float32),
            pltpu.VMEM((128, 128), jnp.float32),
            pltpu.SemaphoreType.DMA,
            pltpu.SemaphoreType.DMA,
            pltpu.SemaphoreType.DMA,
            pltpu.SemaphoreType.DMA,
        ],
        compiler_params=pltpu.CompilerParams(needs_layout_passes=False,
                                             use_tc_tiling_on_sc=True),
        name="item_model_sc_transpose",
    )(_sc_transpose_body)
    return f(item_tt, tail)


def _sc_item_body(item_i, item_t, item_o, idx_v, pair_a, pair_b, sem_a, sem_b):
    wid = lax.axis_index("s") * NC + lax.axis_index("c")
    rbase = wid * NCH
    obase = wid * BPW
    pltpu.sync_copy(item_i.at[pl.ds(rbase, NCH)], idx_v)
    bufs = [pair_a, pair_b]
    sems = [sem_a, sem_b]
    copies = [None, None]

    def fire(j):
        copies[j % 2] = pltpu.async_copy(
            item_t.at[idx_v.at[j]], bufs[j % 2], sems[j % 2])

    fire(0)
    fire(1)
    for j in range(NCH):
        copies[j % 2].wait()
        pltpu.sync_copy(bufs[j % 2], item_o.at[pl.ds(obase + j * CH, CH)])
        if j + 2 < NCH:
            fire(j + 2)


def _sc_body(bus_i, typ_i, sub_i, price_h, bnd_h,
             bus_t, typ_t, sub_t, price_t,
             bus_o, typ_o, sub_o, price_o,
             idx_v, price_v, bnd_v, buf_a, buf_b, sem_a, sem_b):
    wid = lax.axis_index("s") * NC + lax.axis_index("c")
    rbase = wid * NCH
    obase = wid * BPW

    pltpu.sync_copy(bus_i.at[pl.ds(rbase, NCH)], idx_v.at[pl.ds(0, NCH)])
    pltpu.sync_copy(typ_i.at[pl.ds(rbase, NCH)], idx_v.at[pl.ds(NCH, NCH)])
    pltpu.sync_copy(sub_i.at[pl.ds(rbase, NCH)], idx_v.at[pl.ds(2 * NCH, NCH)])
    pltpu.sync_copy(price_h.at[pl.ds(rbase, NCH)], price_v)
    pltpu.sync_copy(bnd_h, bnd_v)

    tabs = [bus_t, typ_t, sub_t, price_t]
    outs = [bus_o, typ_o, sub_o, price_o]
    bufs = [buf_a, buf_b]
    sems = [sem_a, sem_b]
    copies = [None, None]
    n_units = 4 * NCH

    def fire(k):
        t, j = divmod(k, NCH)
        copies[k % 2] = pltpu.async_copy(
            tabs[t].at[idx_v.at[t * NCH + j]], bufs[k % 2], sems[k % 2])

    fire(0)
    fire(1)

    for r in range(NCH):
        for c in range(CH // 16):
            v = price_v[r, pl.ds(c * 16, 16)]
            base = jnp.zeros((16,), jnp.int32)
            n = NBND
            while n > 1:
                half = n // 2
                probe = plsc.load_gather(bnd_v, [base + (half - 1)])
                base = base + jnp.where(probe < v, half, 0)
                n -= half
            probe = plsc.load_gather(bnd_v, [base])
            base = base + jnp.where(probe < v, 1, 0)
            idx_v[3 * NCH + r, pl.ds(c * 16, 16)] = base

    for k in range(n_units):
        copies[k % 2].wait()
        t, j = divmod(k, NCH)
        pltpu.sync_copy(bufs[k % 2], outs[t].at[pl.ds(obase + j * CH, CH)])
        if k + 2 < n_units:
            fire(k + 2)


def _sc_item_gather(item_i, item_t):
    f = functools.partial(
        pl.kernel,
        out_type=jax.ShapeDtypeStruct((B, 2 * EMB), jnp.float32),
        mesh=plsc.VectorSubcoreMesh(core_axis_name="c", subcore_axis_name="s"),
        scratch_types=[
            pltpu.VMEM((NCH, CH), jnp.int32),
            pltpu.VMEM((CH, 2 * EMB), jnp.float32),
            pltpu.VMEM((CH, 2 * EMB), jnp.float32),
            pltpu.SemaphoreType.DMA,
            pltpu.SemaphoreType.DMA,
        ],
        compiler_params=pltpu.CompilerParams(needs_layout_passes=False,
                                             use_tc_tiling_on_sc=True),
        name="item_model_sc_item_gather",
    )(_sc_item_body)
    return f(item_i, item_t)


def _sc_gather(bus_i, typ_i, sub_i, price_i, bnd,
               bus_t, typ_t, sub_t, price_t):
    row = jax.ShapeDtypeStruct((B, EMB), jnp.float32)
    f = functools.partial(
        pl.kernel,
        out_type=[row] * 4,
        mesh=plsc.VectorSubcoreMesh(core_axis_name="c", subcore_axis_name="s"),
        scratch_types=[
            pltpu.VMEM((4 * NCH, CH), jnp.int32),
            pltpu.VMEM((NCH, CH), jnp.float32),
            pltpu.VMEM((NBND,), jnp.float32),
            pltpu.VMEM((CH, EMB), jnp.float32),
            pltpu.VMEM((CH, EMB), jnp.float32),
            pltpu.SemaphoreType.DMA,
            pltpu.SemaphoreType.DMA,
        ],
        compiler_params=pltpu.CompilerParams(needs_layout_passes=False,
                                             use_tc_tiling_on_sc=False),
        name="item_model_sc_gather",
    )(_sc_body)
    return f(bus_i, typ_i, sub_i, price_i, bnd, bus_t, typ_t, sub_t, price_t)


def _tc_body(pair_r, par_r, bus_r, typ_r, sub_r, price_r, img_r,
             wc_r, bc_r, wd_r, bd_r, out_r):
    p = par_r[...]
    item = pair_r[:, 0:EMB] * (1.0 - p) + pair_r[:, EMB:2 * EMB] * p
    attrs = jnp.concatenate([bus_r[...], typ_r[...], sub_r[...]], axis=1)
    u = jnp.dot(attrs, wc_r[...], preferred_element_type=jnp.float32) + bc_r[...]
    cross = attrs * u + attrs
    img = jnp.dot(img_r[...], wd_r[...], preferred_element_type=jnp.float32)
    img = jnp.maximum(img + bd_r[...], 0.0)
    out_r[...] = jnp.concatenate([item, cross, price_r[...], img], axis=1)


def _tc_combine(pair_r, par, bus_r, typ_r, sub_r, price_r, img,
                cross_W, cross_b, dense_W, dense_b):
    blk = 1024
    grid = B // blk
    rows = pl.BlockSpec((blk, EMB), lambda i: (i, 0))
    return pl.pallas_call(
        _tc_body,
        grid=(grid,),
        in_specs=[
            pl.BlockSpec((blk, 2 * EMB), lambda i: (i, 0)),
            pl.BlockSpec((blk, 1), lambda i: (i, 0)),
            rows, rows, rows, rows,
            pl.BlockSpec((blk, 12), lambda i: (i, 0)),
            pl.BlockSpec((3 * EMB, 3 * EMB), lambda i: (0, 0)),
            pl.BlockSpec((1, 3 * EMB), lambda i: (0, 0)),
            pl.BlockSpec((12, 12), lambda i: (0, 0)),
            pl.BlockSpec((1, 12), lambda i: (0, 0)),
        ],
        out_specs=pl.BlockSpec((blk, 332), lambda i: (i, 0)),
        out_shape=jax.ShapeDtypeStruct((B, 332), jnp.float32),
    )(pair_r, par, bus_r, typ_r, sub_r, price_r, img,
      cross_W, cross_b, dense_W, dense_b)


def kernel(last_product_id, last_product_business_desc, last_product_type_desc,
           last_product_sub_category, last_product_list_price,
           last_image_embedding_pca, item_table, business_table, type_table,
           subcat_table, price_table, price_boundaries, cross_W, cross_b,
           dense_W, dense_b):
    right = last_product_id >= PAIR_SPLIT
    item_i = jnp.where(right, last_product_id - PAIR_H,
                       last_product_id).reshape(IR, CH)
    bus_i = last_product_business_desc.reshape(IR, CH)
    typ_i = last_product_type_desc.reshape(IR, CH)
    sub_i = last_product_sub_category.reshape(IR, CH)
    price_i = last_product_list_price.reshape(IR, CH)
    bnd = jnp.concatenate(
        [price_boundaries,
         jnp.full((NBND - price_boundaries.shape[0],), jnp.inf, jnp.float32)])
    item_tt = item_table.T
    tail = jax.lax.slice(item_tt, (0, ITEM_V - EMB), (EMB, ITEM_V))
    pair_table = _sc_transpose(item_tt, tail)
    par = right.astype(jnp.float32).reshape(B, 1)
    pair_r = _sc_item_gather(item_i, pair_table)
    bus_r, typ_r, sub_r, price_r = _sc_gather(
        bus_i, typ_i, sub_i, price_i, bnd,
        business_table, type_table, subcat_table, price_table)
    return _tc_combine(pair_r, par, bus_r, typ_r, sub_r, price_r,
                       last_image_embedding_pca, cross_W,
                       cross_b.reshape(1, 3 * EMB), dense_W,
                       dense_b.reshape(1, 12))


# speedup vs baseline: 2.8574x; 2.8574x over previous
import functools

import jax
import jax.numpy as jnp
from jax import lax
from jax.experimental import pallas as pl
from jax.experimental.pallas import tpu as pltpu
from jax.experimental.pallas import tpu_sc as plsc

B = 16384
EMB = 64
ITEM_V = 1000000
PAIR_H = 499712
PAIR_R = 500288
PAIR_SPLIT = PAIR_R
TBLK = 1024
NC = 2
NS = 16
NW = NC * NS
BPW = B // NW
CH = 128
NCH = BPW // CH
IR = B // CH
NBND = 1024


def _tc_transpose_body(l_ref, r_ref, out_ref):
    out_ref[...] = jnp.concatenate([l_ref[...].T, r_ref[...].T], axis=1)


def _tc_transpose(item_tt):
    grid = (PAIR_R + TBLK - 1) // TBLK
    return pl.pallas_call(
        _tc_transpose_body,
        grid=(grid,),
        in_specs=[
            pl.BlockSpec((EMB, TBLK), lambda i: (0, i)),
            pl.BlockSpec((EMB, TBLK), lambda i: (0, PAIR_H // TBLK + i)),
        ],
        out_specs=pl.BlockSpec((TBLK, 2 * EMB), lambda i: (i, 0)),
        out_shape=jax.ShapeDtypeStruct((PAIR_R, 2 * EMB), jnp.float32),
    )(item_tt, item_tt)


def _sc_item_body(item_i, item_t, item_o, idx_v, pair_a, pair_b, sem_a, sem_b):
    wid = lax.axis_index("s") * NC + lax.axis_index("c")
    rbase = wid * NCH
    obase = wid * BPW
    pltpu.sync_copy(item_i.at[pl.ds(rbase, NCH)], idx_v)
    bufs = [pair_a, pair_b]
    sems = [sem_a, sem_b]
    copies = [None, None]

    def fire(j):
        copies[j % 2] = pltpu.async_copy(
            item_t.at[idx_v.at[j]], bufs[j % 2], sems[j % 2])

    fire(0)
    fire(1)
    for j in range(NCH):
        copies[j % 2].wait()
        pltpu.sync_copy(bufs[j % 2], item_o.at[pl.ds(obase + j * CH, CH)])
        if j + 2 < NCH:
            fire(j + 2)


def _sc_body(bus_i, typ_i, sub_i, price_h, bnd_h,
             bus_t, typ_t, sub_t, price_t,
             bus_o, typ_o, sub_o, price_o,
             idx_v, price_v, bnd_v, buf_a, buf_b, sem_a, sem_b):
    wid = lax.axis_index("s") * NC + lax.axis_index("c")
    rbase = wid * NCH
    obase = wid * BPW

    pltpu.sync_copy(bus_i.at[pl.ds(rbase, NCH)], idx_v.at[pl.ds(0, NCH)])
    pltpu.sync_copy(typ_i.at[pl.ds(rbase, NCH)], idx_v.at[pl.ds(NCH, NCH)])
    pltpu.sync_copy(sub_i.at[pl.ds(rbase, NCH)], idx_v.at[pl.ds(2 * NCH, NCH)])
    pltpu.sync_copy(price_h.at[pl.ds(rbase, NCH)], price_v)
    pltpu.sync_copy(bnd_h, bnd_v)

    tabs = [bus_t, typ_t, sub_t, price_t]
    outs = [bus_o, typ_o, sub_o, price_o]
    bufs = [buf_a, buf_b]
    sems = [sem_a, sem_b]
    copies = [None, None]
    n_units = 4 * NCH

    def fire(k):
        t, j = divmod(k, NCH)
        copies[k % 2] = pltpu.async_copy(
            tabs[t].at[idx_v.at[t * NCH + j]], bufs[k % 2], sems[k % 2])

    fire(0)
    fire(1)

    for r in range(NCH):
        for c in range(CH // 16):
            v = price_v[r, pl.ds(c * 16, 16)]
            base = jnp.zeros((16,), jnp.int32)
            n = NBND
            while n > 1:
                half = n // 2
                probe = plsc.load_gather(bnd_v, [base + (half - 1)])
                base = base + jnp.where(probe < v, half, 0)
                n -= half
            probe = plsc.load_gather(bnd_v, [base])
            base = base + jnp.where(probe < v, 1, 0)
            idx_v[3 * NCH + r, pl.ds(c * 16, 16)] = base

    for k in range(n_units):
        copies[k % 2].wait()
        t, j = divmod(k, NCH)
        pltpu.sync_copy(bufs[k % 2], outs[t].at[pl.ds(obase + j * CH, CH)])
        if k + 2 < n_units:
            fire(k + 2)


def _sc_item_gather(item_i, item_t):
    f = functools.partial(
        pl.kernel,
        out_type=jax.ShapeDtypeStruct((B, 2 * EMB), jnp.float32),
        mesh=plsc.VectorSubcoreMesh(core_axis_name="c", subcore_axis_name="s"),
        scratch_types=[
            pltpu.VMEM((NCH, CH), jnp.int32),
            pltpu.VMEM((CH, 2 * EMB), jnp.float32),
            pltpu.VMEM((CH, 2 * EMB), jnp.float32),
            pltpu.SemaphoreType.DMA,
            pltpu.SemaphoreType.DMA,
        ],
        compiler_params=pltpu.CompilerParams(needs_layout_passes=False,
                                             use_tc_tiling_on_sc=True),
        name="item_model_sc_item_gather",
    )(_sc_item_body)
    return f(item_i, item_t)


def _sc_gather(bus_i, typ_i, sub_i, price_i, bnd,
               bus_t, typ_t, sub_t, price_t):
    row = jax.ShapeDtypeStruct((B, EMB), jnp.float32)
    f = functools.partial(
        pl.kernel,
        out_type=[row] * 4,
        mesh=plsc.VectorSubcoreMesh(core_axis_name="c", subcore_axis_name="s"),
        scratch_types=[
            pltpu.VMEM((4 * NCH, CH), jnp.int32),
            pltpu.VMEM((NCH, CH), jnp.float32),
            pltpu.VMEM((NBND,), jnp.float32),
            pltpu.VMEM((CH, EMB), jnp.float32),
            pltpu.VMEM((CH, EMB), jnp.float32),
            pltpu.SemaphoreType.DMA,
            pltpu.SemaphoreType.DMA,
        ],
        compiler_params=pltpu.CompilerParams(needs_layout_passes=False,
                                             use_tc_tiling_on_sc=False),
        name="item_model_sc_gather",
    )(_sc_body)
    return f(bus_i, typ_i, sub_i, price_i, bnd, bus_t, typ_t, sub_t, price_t)


def _tc_body(pair_r, par_r, bus_r, typ_r, sub_r, price_r, img_r,
             wc_r, bc_r, wd_r, bd_r, out_r):
    p = par_r[...]
    item = pair_r[:, 0:EMB] * (1.0 - p) + pair_r[:, EMB:2 * EMB] * p
    attrs = jnp.concatenate([bus_r[...], typ_r[...], sub_r[...]], axis=1)
    u = jnp.dot(attrs, wc_r[...], preferred_element_type=jnp.float32) + bc_r[...]
    cross = attrs * u + attrs
    img = jnp.dot(img_r[...], wd_r[...], preferred_element_type=jnp.float32)
    img = jnp.maximum(img + bd_r[...], 0.0)
    out_r[...] = jnp.concatenate([item, cross, price_r[...], img], axis=1)


def _tc_combine(pair_r, par, bus_r, typ_r, sub_r, price_r, img,
                cross_W, cross_b, dense_W, dense_b):
    blk = 1024
    grid = B // blk
    rows = pl.BlockSpec((blk, EMB), lambda i: (i, 0))
    return pl.pallas_call(
        _tc_body,
        grid=(grid,),
        in_specs=[
            pl.BlockSpec((blk, 2 * EMB), lambda i: (i, 0)),
            pl.BlockSpec((blk, 1), lambda i: (i, 0)),
            rows, rows, rows, rows,
            pl.BlockSpec((blk, 12), lambda i: (i, 0)),
            pl.BlockSpec((3 * EMB, 3 * EMB), lambda i: (0, 0)),
            pl.BlockSpec((1, 3 * EMB), lambda i: (0, 0)),
            pl.BlockSpec((12, 12), lambda i: (0, 0)),
            pl.BlockSpec((1, 12), lambda i: (0, 0)),
        ],
        out_specs=pl.BlockSpec((blk, 332), lambda i: (i, 0)),
        out_shape=jax.ShapeDtypeStruct((B, 332), jnp.float32),
    )(pair_r, par, bus_r, typ_r, sub_r, price_r, img,
      cross_W, cross_b, dense_W, dense_b)


def kernel(last_product_id, last_product_business_desc, last_product_type_desc,
           last_product_sub_category, last_product_list_price,
           last_image_embedding_pca, item_table, business_table, type_table,
           subcat_table, price_table, price_boundaries, cross_W, cross_b,
           dense_W, dense_b):
    right = last_product_id >= PAIR_SPLIT
    item_i = jnp.where(right, last_product_id - PAIR_H,
                       last_product_id).reshape(IR, CH)
    bus_i = last_product_business_desc.reshape(IR, CH)
    typ_i = last_product_type_desc.reshape(IR, CH)
    sub_i = last_product_sub_category.reshape(IR, CH)
    price_i = last_product_list_price.reshape(IR, CH)
    bnd = jnp.concatenate(
        [price_boundaries,
         jnp.full((NBND - price_boundaries.shape[0],), jnp.inf, jnp.float32)])
    item_tt = item_table.T
    pair_table = _tc_transpose(item_tt)
    par = right.astype(jnp.float32).reshape(B, 1)
    pair_r = _sc_item_gather(item_i, pair_table)
    bus_r, typ_r, sub_r, price_r = _sc_gather(
        bus_i, typ_i, sub_i, price_i, bnd,
        business_table, type_table, subcat_table, price_table)
    return _tc_combine(pair_r, par, bus_r, typ_r, sub_r, price_r,
                       last_image_embedding_pca, cross_W,
                       cross_b.reshape(1, 3 * EMB), dense_W,
                       dense_b.reshape(1, 12))


# speedup vs baseline: 4.2411x; 1.4842x over previous
import functools

import jax
import jax.numpy as jnp
from jax import lax
from jax.experimental import pallas as pl
from jax.experimental.pallas import tpu as pltpu
from jax.experimental.pallas import tpu_sc as plsc

B = 16384
EMB = 64
ITEM_V = 1000000
PAIR_H = 499712
PAIR_R = 500288
PAIR_SPLIT = PAIR_R
TBLK = 4096
NC = 2
NS = 16
NW = NC * NS
BPW = B // NW
CH = 128
NCH = BPW // CH
IR = B // CH
NBND = 1024


def _tc_transpose_body(l_ref, r_ref, out_ref):
    out_ref[...] = jnp.concatenate([l_ref[...].T, r_ref[...].T], axis=1)


def _tc_transpose(item_tt):
    grid = (PAIR_R + TBLK - 1) // TBLK
    return pl.pallas_call(
        _tc_transpose_body,
        grid=(grid,),
        in_specs=[
            pl.BlockSpec((EMB, TBLK), lambda i: (0, i)),
            pl.BlockSpec((EMB, TBLK), lambda i: (0, PAIR_H // TBLK + i)),
        ],
        out_specs=pl.BlockSpec((TBLK, 2 * EMB), lambda i: (i, 0)),
        out_shape=jax.ShapeDtypeStruct((PAIR_R, 2 * EMB), jnp.float32),
    )(item_tt, item_tt)


def _sc_item_body(item_i, item_t, item_o, idx_v, pair_a, pair_b, sem_a, sem_b):
    wid = lax.axis_index("s") * NC + lax.axis_index("c")
    rbase = wid * NCH
    obase = wid * BPW
    pltpu.sync_copy(item_i.at[pl.ds(rbase, NCH)], idx_v)
    bufs = [pair_a, pair_b]
    sems = [sem_a, sem_b]
    copies = [None, None]

    def fire(j):
        copies[j % 2] = pltpu.async_copy(
            item_t.at[idx_v.at[j]], bufs[j % 2], sems[j % 2])

    fire(0)
    fire(1)
    for j in range(NCH):
        copies[j % 2].wait()
        pltpu.sync_copy(bufs[j % 2], item_o.at[pl.ds(obase + j * CH, CH)])
        if j + 2 < NCH:
            fire(j + 2)


def _sc_body(bus_i, typ_i, sub_i, price_h, bnd_h,
             bus_t, typ_t, sub_t, price_t,
             bus_o, typ_o, sub_o, price_o,
             idx_v, price_v, bnd_v, buf_a, buf_b, sem_a, sem_b):
    wid = lax.axis_index("s") * NC + lax.axis_index("c")
    rbase = wid * NCH
    obase = wid * BPW

    pltpu.sync_copy(bus_i.at[pl.ds(rbase, NCH)], idx_v.at[pl.ds(0, NCH)])
    pltpu.sync_copy(typ_i.at[pl.ds(rbase, NCH)], idx_v.at[pl.ds(NCH, NCH)])
    pltpu.sync_copy(sub_i.at[pl.ds(rbase, NCH)], idx_v.at[pl.ds(2 * NCH, NCH)])
    pltpu.sync_copy(price_h.at[pl.ds(rbase, NCH)], price_v)
    pltpu.sync_copy(bnd_h, bnd_v)

    tabs = [bus_t, typ_t, sub_t, price_t]
    outs = [bus_o, typ_o, sub_o, price_o]
    bufs = [buf_a, buf_b]
    sems = [sem_a, sem_b]
    copies = [None, None]
    n_units = 4 * NCH

    def fire(k):
        t, j = divmod(k, NCH)
        copies[k % 2] = pltpu.async_copy(
            tabs[t].at[idx_v.at[t * NCH + j]], bufs[k % 2], sems[k % 2])

    fire(0)
    fire(1)

    for r in range(NCH):
        for c in range(CH // 16):
            v = price_v[r, pl.ds(c * 16, 16)]
            base = jnp.zeros((16,), jnp.int32)
            n = NBND
            while n > 1:
                half = n // 2
                probe = plsc.load_gather(bnd_v, [base + (half - 1)])
                base = base + jnp.where(probe < v, half, 0)
                n -= half
            probe = plsc.load_gather(bnd_v, [base])
            base = base + jnp.where(probe < v, 1, 0)
            idx_v[3 * NCH + r, pl.ds(c * 16, 16)] = base

    for k in range(n_units):
        copies[k % 2].wait()
        t, j = divmod(k, NCH)
        pltpu.sync_copy(bufs[k % 2], outs[t].at[pl.ds(obase + j * CH, CH)])
        if k + 2 < n_units:
            fire(k + 2)


def _sc_item_gather(item_i, item_t):
    f = functools.partial(
        pl.kernel,
        out_type=jax.ShapeDtypeStruct((B, 2 * EMB), jnp.float32),
        mesh=plsc.VectorSubcoreMesh(core_axis_name="c", subcore_axis_name="s"),
        scratch_types=[
            pltpu.VMEM((NCH, CH), jnp.int32),
            pltpu.VMEM((CH, 2 * EMB), jnp.float32),
            pltpu.VMEM((CH, 2 * EMB), jnp.float32),
            pltpu.SemaphoreType.DMA,
            pltpu.SemaphoreType.DMA,
        ],
        compiler_params=pltpu.CompilerParams(needs_layout_passes=False,
                                             use_tc_tiling_on_sc=True),
        name="item_model_sc_item_gather",
    )(_sc_item_body)
    return f(item_i, item_t)


def _sc_gather(bus_i, typ_i, sub_i, price_i, bnd,
               bus_t, typ_t, sub_t, price_t):
    row = jax.ShapeDtypeStruct((B, EMB), jnp.float32)
    f = functools.partial(
        pl.kernel,
        out_type=[row] * 4,
        mesh=plsc.VectorSubcoreMesh(core_axis_name="c", subcore_axis_name="s"),
        scratch_types=[
            pltpu.VMEM((4 * NCH, CH), jnp.int32),
            pltpu.VMEM((NCH, CH), jnp.float32),
            pltpu.VMEM((NBND,), jnp.float32),
            pltpu.VMEM((CH, EMB), jnp.float32),
            pltpu.VMEM((CH, EMB), jnp.float32),
            pltpu.SemaphoreType.DMA,
            pltpu.SemaphoreType.DMA,
        ],
        compiler_params=pltpu.CompilerParams(needs_layout_passes=False,
                                             use_tc_tiling_on_sc=False),
        name="item_model_sc_gather",
    )(_sc_body)
    return f(bus_i, typ_i, sub_i, price_i, bnd, bus_t, typ_t, sub_t, price_t)


def _tc_body(pair_r, par_r, bus_r, typ_r, sub_r, price_r, img_r,
             wc_r, bc_r, wd_r, bd_r, out_r):
    p = par_r[...]
    item = pair_r[:, 0:EMB] * (1.0 - p) + pair_r[:, EMB:2 * EMB] * p
    attrs = jnp.concatenate([bus_r[...], typ_r[...], sub_r[...]], axis=1)
    u = jnp.dot(attrs, wc_r[...], preferred_element_type=jnp.float32) + bc_r[...]
    cross = attrs * u + attrs
    img = jnp.dot(img_r[...], wd_r[...], preferred_element_type=jnp.float32)
    img = jnp.maximum(img + bd_r[...], 0.0)
    out_r[...] = jnp.concatenate([item, cross, price_r[...], img], axis=1)


def _tc_combine(pair_r, par, bus_r, typ_r, sub_r, price_r, img,
                cross_W, cross_b, dense_W, dense_b):
    blk = 1024
    grid = B // blk
    rows = pl.BlockSpec((blk, EMB), lambda i: (i, 0))
    return pl.pallas_call(
        _tc_body,
        grid=(grid,),
        in_specs=[
            pl.BlockSpec((blk, 2 * EMB), lambda i: (i, 0)),
            pl.BlockSpec((blk, 1), lambda i: (i, 0)),
            rows, rows, rows, rows,
            pl.BlockSpec((blk, 12), lambda i: (i, 0)),
            pl.BlockSpec((3 * EMB, 3 * EMB), lambda i: (0, 0)),
            pl.BlockSpec((1, 3 * EMB), lambda i: (0, 0)),
            pl.BlockSpec((12, 12), lambda i: (0, 0)),
            pl.BlockSpec((1, 12), lambda i: (0, 0)),
        ],
        out_specs=pl.BlockSpec((blk, 332), lambda i: (i, 0)),
        out_shape=jax.ShapeDtypeStruct((B, 332), jnp.float32),
    )(pair_r, par, bus_r, typ_r, sub_r, price_r, img,
      cross_W, cross_b, dense_W, dense_b)


def kernel(last_product_id, last_product_business_desc, last_product_type_desc,
           last_product_sub_category, last_product_list_price,
           last_image_embedding_pca, item_table, business_table, type_table,
           subcat_table, price_table, price_boundaries, cross_W, cross_b,
           dense_W, dense_b):
    right = last_product_id >= PAIR_SPLIT
    item_i = jnp.where(right, last_product_id - PAIR_H,
                       last_product_id).reshape(IR, CH)
    bus_i = last_product_business_desc.reshape(IR, CH)
    typ_i = last_product_type_desc.reshape(IR, CH)
    sub_i = last_product_sub_category.reshape(IR, CH)
    price_i = last_product_list_price.reshape(IR, CH)
    bnd = jnp.concatenate(
        [price_boundaries,
         jnp.full((NBND - price_boundaries.shape[0],), jnp.inf, jnp.float32)])
    item_tt = item_table.T
    pair_table = _tc_transpose(item_tt)
    par = right.astype(jnp.float32).reshape(B, 1)
    pair_r = _sc_item_gather(item_i, pair_table)
    bus_r, typ_r, sub_r, price_r = _sc_gather(
        bus_i, typ_i, sub_i, price_i, bnd,
        business_table, type_table, subcat_table, price_table)
    return _tc_combine(pair_r, par, bus_r, typ_r, sub_r, price_r,
                       last_image_embedding_pca, cross_W,
                       cross_b.reshape(1, 3 * EMB), dense_W,
                       dense_b.reshape(1, 12))


# speedup vs baseline: 4.6024x; 1.0852x over previous
import functools

import jax
import jax.numpy as jnp
from jax import lax
from jax.experimental import pallas as pl
from jax.experimental.pallas import tpu as pltpu
from jax.experimental.pallas import tpu_sc as plsc

B = 16384
EMB = 64
ITEM_V = 1000000
PAIR_H = 499712
PAIR_R = 500288
PAIR_SPLIT = PAIR_R
TBLK = 8192
NC = 2
NS = 16
NW = NC * NS
BPW = B // NW
CH = 128
NCH = BPW // CH
IR = B // CH
NBND = 1024


def _tc_transpose_body(l_ref, r_ref, out_ref):
    out_ref[...] = jnp.concatenate([l_ref[...].T, r_ref[...].T], axis=1)


def _tc_transpose(item_tt):
    grid = (PAIR_R + TBLK - 1) // TBLK
    return pl.pallas_call(
        _tc_transpose_body,
        grid=(grid,),
        in_specs=[
            pl.BlockSpec((EMB, TBLK), lambda i: (0, i)),
            pl.BlockSpec((EMB, TBLK), lambda i: (0, PAIR_H // TBLK + i)),
        ],
        out_specs=pl.BlockSpec((TBLK, 2 * EMB), lambda i: (i, 0)),
        out_shape=jax.ShapeDtypeStruct((PAIR_R, 2 * EMB), jnp.float32),
    )(item_tt, item_tt)


def _sc_item_body(item_i, item_t, item_o, idx_v, pair_a, pair_b, sem_a, sem_b):
    wid = lax.axis_index("s") * NC + lax.axis_index("c")
    rbase = wid * NCH
    obase = wid * BPW
    pltpu.sync_copy(item_i.at[pl.ds(rbase, NCH)], idx_v)
    bufs = [pair_a, pair_b]
    sems = [sem_a, sem_b]
    copies = [None, None]

    def fire(j):
        copies[j % 2] = pltpu.async_copy(
            item_t.at[idx_v.at[j]], bufs[j % 2], sems[j % 2])

    fire(0)
    fire(1)
    for j in range(NCH):
        copies[j % 2].wait()
        pltpu.sync_copy(bufs[j % 2], item_o.at[pl.ds(obase + j * CH, CH)])
        if j + 2 < NCH:
            fire(j + 2)


def _sc_body(bus_i, typ_i, sub_i, price_h, bnd_h,
             bus_t, typ_t, sub_t, price_t,
             bus_o, typ_o, sub_o, price_o,
             idx_v, price_v, bnd_v, buf_a, buf_b, sem_a, sem_b):
    wid = lax.axis_index("s") * NC + lax.axis_index("c")
    rbase = wid * NCH
    obase = wid * BPW

    pltpu.sync_copy(bus_i.at[pl.ds(rbase, NCH)], idx_v.at[pl.ds(0, NCH)])
    pltpu.sync_copy(typ_i.at[pl.ds(rbase, NCH)], idx_v.at[pl.ds(NCH, NCH)])
    pltpu.sync_copy(sub_i.at[pl.ds(rbase, NCH)], idx_v.at[pl.ds(2 * NCH, NCH)])
    pltpu.sync_copy(price_h.at[pl.ds(rbase, NCH)], price_v)
    pltpu.sync_copy(bnd_h, bnd_v)

    tabs = [bus_t, typ_t, sub_t, price_t]
    outs = [bus_o, typ_o, sub_o, price_o]
    bufs = [buf_a, buf_b]
    sems = [sem_a, sem_b]
    copies = [None, None]
    n_units = 4 * NCH

    def fire(k):
        t, j = divmod(k, NCH)
        copies[k % 2] = pltpu.async_copy(
            tabs[t].at[idx_v.at[t * NCH + j]], bufs[k % 2], sems[k % 2])

    fire(0)
    fire(1)

    for r in range(NCH):
        for c in range(CH // 16):
            v = price_v[r, pl.ds(c * 16, 16)]
            base = jnp.zeros((16,), jnp.int32)
            n = NBND
            while n > 1:
                half = n // 2
                probe = plsc.load_gather(bnd_v, [base + (half - 1)])
                base = base + jnp.where(probe < v, half, 0)
                n -= half
            probe = plsc.load_gather(bnd_v, [base])
            base = base + jnp.where(probe < v, 1, 0)
            idx_v[3 * NCH + r, pl.ds(c * 16, 16)] = base

    for k in range(n_units):
        copies[k % 2].wait()
        t, j = divmod(k, NCH)
        pltpu.sync_copy(bufs[k % 2], outs[t].at[pl.ds(obase + j * CH, CH)])
        if k + 2 < n_units:
            fire(k + 2)


def _sc_item_gather(item_i, item_t):
    f = functools.partial(
        pl.kernel,
        out_type=jax.ShapeDtypeStruct((B, 2 * EMB), jnp.float32),
        mesh=plsc.VectorSubcoreMesh(core_axis_name="c", subcore_axis_name="s"),
        scratch_types=[
            pltpu.VMEM((NCH, CH), jnp.int32),
            pltpu.VMEM((CH, 2 * EMB), jnp.float32),
            pltpu.VMEM((CH, 2 * EMB), jnp.float32),
            pltpu.SemaphoreType.DMA,
            pltpu.SemaphoreType.DMA,
        ],
        compiler_params=pltpu.CompilerParams(needs_layout_passes=False,
                                             use_tc_tiling_on_sc=True),
        name="item_model_sc_item_gather",
    )(_sc_item_body)
    return f(item_i, item_t)


def _sc_gather(bus_i, typ_i, sub_i, price_i, bnd,
               bus_t, typ_t, sub_t, price_t):
    row = jax.ShapeDtypeStruct((B, EMB), jnp.float32)
    f = functools.partial(
        pl.kernel,
        out_type=[row] * 4,
        mesh=plsc.VectorSubcoreMesh(core_axis_name="c", subcore_axis_name="s"),
        scratch_types=[
            pltpu.VMEM((4 * NCH, CH), jnp.int32),
            pltpu.VMEM((NCH, CH), jnp.float32),
            pltpu.VMEM((NBND,), jnp.float32),
            pltpu.VMEM((CH, EMB), jnp.float32),
            pltpu.VMEM((CH, EMB), jnp.float32),
            pltpu.SemaphoreType.DMA,
            pltpu.SemaphoreType.DMA,
        ],
        compiler_params=pltpu.CompilerParams(needs_layout_passes=False,
                                             use_tc_tiling_on_sc=False),
        name="item_model_sc_gather",
    )(_sc_body)
    return f(bus_i, typ_i, sub_i, price_i, bnd, bus_t, typ_t, sub_t, price_t)


def _tc_body(pair_r, par_r, bus_r, typ_r, sub_r, price_r, img_r,
             wc_r, bc_r, wd_r, bd_r, out_r):
    p = par_r[...]
    item = pair_r[:, 0:EMB] * (1.0 - p) + pair_r[:, EMB:2 * EMB] * p
    attrs = jnp.concatenate([bus_r[...], typ_r[...], sub_r[...]], axis=1)
    u = jnp.dot(attrs, wc_r[...], preferred_element_type=jnp.float32) + bc_r[...]
    cross = attrs * u + attrs
    img = jnp.dot(img_r[...], wd_r[...], preferred_element_type=jnp.float32)
    img = jnp.maximum(img + bd_r[...], 0.0)
    out_r[...] = jnp.concatenate([item, cross, price_r[...], img], axis=1)


def _tc_combine(pair_r, par, bus_r, typ_r, sub_r, price_r, img,
                cross_W, cross_b, dense_W, dense_b):
    blk = 1024
    grid = B // blk
    rows = pl.BlockSpec((blk, EMB), lambda i: (i, 0))
    return pl.pallas_call(
        _tc_body,
        grid=(grid,),
        in_specs=[
            pl.BlockSpec((blk, 2 * EMB), lambda i: (i, 0)),
            pl.BlockSpec((blk, 1), lambda i: (i, 0)),
            rows, rows, rows, rows,
            pl.BlockSpec((blk, 12), lambda i: (i, 0)),
            pl.BlockSpec((3 * EMB, 3 * EMB), lambda i: (0, 0)),
            pl.BlockSpec((1, 3 * EMB), lambda i: (0, 0)),
            pl.BlockSpec((12, 12), lambda i: (0, 0)),
            pl.BlockSpec((1, 12), lambda i: (0, 0)),
        ],
        out_specs=pl.BlockSpec((blk, 332), lambda i: (i, 0)),
        out_shape=jax.ShapeDtypeStruct((B, 332), jnp.float32),
    )(pair_r, par, bus_r, typ_r, sub_r, price_r, img,
      cross_W, cross_b, dense_W, dense_b)


def kernel(last_product_id, last_product_business_desc, last_product_type_desc,
           last_product_sub_category, last_product_list_price,
           last_image_embedding_pca, item_table, business_table, type_table,
           subcat_table, price_table, price_boundaries, cross_W, cross_b,
           dense_W, dense_b):
    right = last_product_id >= PAIR_SPLIT
    item_i = jnp.where(right, last_product_id - PAIR_H,
                       last_product_id).reshape(IR, CH)
    bus_i = last_product_business_desc.reshape(IR, CH)
    typ_i = last_product_type_desc.reshape(IR, CH)
    sub_i = last_product_sub_category.reshape(IR, CH)
    price_i = last_product_list_price.reshape(IR, CH)
    bnd = jnp.concatenate(
        [price_boundaries,
         jnp.full((NBND - price_boundaries.shape[0],), jnp.inf, jnp.float32)])
    item_tt = item_table.T
    pair_table = _tc_transpose(item_tt)
    par = right.astype(jnp.float32).reshape(B, 1)
    pair_r = _sc_item_gather(item_i, pair_table)
    bus_r, typ_r, sub_r, price_r = _sc_gather(
        bus_i, typ_i, sub_i, price_i, bnd,
        business_table, type_table, subcat_table, price_table)
    return _tc_combine(pair_r, par, bus_r, typ_r, sub_r, price_r,
                       last_image_embedding_pca, cross_W,
                       cross_b.reshape(1, 3 * EMB), dense_W,
                       dense_b.reshape(1, 12))


# speedup vs baseline: 4.7437x; 1.0307x over previous
import functools

import jax
import jax.numpy as jnp
from jax import lax
from jax.experimental import pallas as pl
from jax.experimental.pallas import tpu as pltpu
from jax.experimental.pallas import tpu_sc as plsc

B = 16384
EMB = 64
ITEM_V = 1000000
PAIR_H = 491520
PAIR_R = 508480
PAIR_SPLIT = PAIR_R
TBLK = 16384
NC = 2
NS = 16
NW = NC * NS
BPW = B // NW
CH = 128
NCH = BPW // CH
IR = B // CH
NBND = 1024


def _tc_transpose_body(l_ref, r_ref, out_ref):
    out_ref[...] = jnp.concatenate([l_ref[...].T, r_ref[...].T], axis=1)


def _tc_transpose(item_tt):
    grid = (PAIR_R + TBLK - 1) // TBLK
    return pl.pallas_call(
        _tc_transpose_body,
        grid=(grid,),
        in_specs=[
            pl.BlockSpec((EMB, TBLK), lambda i: (0, i)),
            pl.BlockSpec((EMB, TBLK), lambda i: (0, PAIR_H // TBLK + i)),
        ],
        out_specs=pl.BlockSpec((TBLK, 2 * EMB), lambda i: (i, 0)),
        out_shape=jax.ShapeDtypeStruct((PAIR_R, 2 * EMB), jnp.float32),
    )(item_tt, item_tt)


def _sc_item_body(item_i, item_t, item_o, idx_v, pair_a, pair_b, sem_a, sem_b):
    wid = lax.axis_index("s") * NC + lax.axis_index("c")
    rbase = wid * NCH
    obase = wid * BPW
    pltpu.sync_copy(item_i.at[pl.ds(rbase, NCH)], idx_v)
    bufs = [pair_a, pair_b]
    sems = [sem_a, sem_b]
    copies = [None, None]

    def fire(j):
        copies[j % 2] = pltpu.async_copy(
            item_t.at[idx_v.at[j]], bufs[j % 2], sems[j % 2])

    fire(0)
    fire(1)
    for j in range(NCH):
        copies[j % 2].wait()
        pltpu.sync_copy(bufs[j % 2], item_o.at[pl.ds(obase + j * CH, CH)])
        if j + 2 < NCH:
            fire(j + 2)


def _sc_body(bus_i, typ_i, sub_i, price_h, bnd_h,
             bus_t, typ_t, sub_t, price_t,
             bus_o, typ_o, sub_o, price_o,
             idx_v, price_v, bnd_v, buf_a, buf_b, sem_a, sem_b):
    wid = lax.axis_index("s") * NC + lax.axis_index("c")
    rbase = wid * NCH
    obase = wid * BPW

    pltpu.sync_copy(bus_i.at[pl.ds(rbase, NCH)], idx_v.at[pl.ds(0, NCH)])
    pltpu.sync_copy(typ_i.at[pl.ds(rbase, NCH)], idx_v.at[pl.ds(NCH, NCH)])
    pltpu.sync_copy(sub_i.at[pl.ds(rbase, NCH)], idx_v.at[pl.ds(2 * NCH, NCH)])
    pltpu.sync_copy(price_h.at[pl.ds(rbase, NCH)], price_v)
    pltpu.sync_copy(bnd_h, bnd_v)

    tabs = [bus_t, typ_t, sub_t, price_t]
    outs = [bus_o, typ_o, sub_o, price_o]
    bufs = [buf_a, buf_b]
    sems = [sem_a, sem_b]
    copies = [None, None]
    n_units = 4 * NCH

    def fire(k):
        t, j = divmod(k, NCH)
        copies[k % 2] = pltpu.async_copy(
            tabs[t].at[idx_v.at[t * NCH + j]], bufs[k % 2], sems[k % 2])

    fire(0)
    fire(1)

    for r in range(NCH):
        for c in range(CH // 16):
            v = price_v[r, pl.ds(c * 16, 16)]
            base = jnp.zeros((16,), jnp.int32)
            n = NBND
            while n > 1:
                half = n // 2
                probe = plsc.load_gather(bnd_v, [base + (half - 1)])
                base = base + jnp.where(probe < v, half, 0)
                n -= half
            probe = plsc.load_gather(bnd_v, [base])
            base = base + jnp.where(probe < v, 1, 0)
            idx_v[3 * NCH + r, pl.ds(c * 16, 16)] = base

    for k in range(n_units):
        copies[k % 2].wait()
        t, j = divmod(k, NCH)
        pltpu.sync_copy(bufs[k % 2], outs[t].at[pl.ds(obase + j * CH, CH)])
        if k + 2 < n_units:
            fire(k + 2)


def _sc_item_gather(item_i, item_t):
    f = functools.partial(
        pl.kernel,
        out_type=jax.ShapeDtypeStruct((B, 2 * EMB), jnp.float32),
        mesh=plsc.VectorSubcoreMesh(core_axis_name="c", subcore_axis_name="s"),
        scratch_types=[
            pltpu.VMEM((NCH, CH), jnp.int32),
            pltpu.VMEM((CH, 2 * EMB), jnp.float32),
            pltpu.VMEM((CH, 2 * EMB), jnp.float32),
            pltpu.SemaphoreType.DMA,
            pltpu.SemaphoreType.DMA,
        ],
        compiler_params=pltpu.CompilerParams(needs_layout_passes=False,
                                             use_tc_tiling_on_sc=True),
        name="item_model_sc_item_gather",
    )(_sc_item_body)
    return f(item_i, item_t)


def _sc_gather(bus_i, typ_i, sub_i, price_i, bnd,
               bus_t, typ_t, sub_t, price_t):
    row = jax.ShapeDtypeStruct((B, EMB), jnp.float32)
    f = functools.partial(
        pl.kernel,
        out_type=[row] * 4,
        mesh=plsc.VectorSubcoreMesh(core_axis_name="c", subcore_axis_name="s"),
        scratch_types=[
            pltpu.VMEM((4 * NCH, CH), jnp.int32),
            pltpu.VMEM((NCH, CH), jnp.float32),
            pltpu.VMEM((NBND,), jnp.float32),
            pltpu.VMEM((CH, EMB), jnp.float32),
            pltpu.VMEM((CH, EMB), jnp.float32),
            pltpu.SemaphoreType.DMA,
            pltpu.SemaphoreType.DMA,
        ],
        compiler_params=pltpu.CompilerParams(needs_layout_passes=False,
                                             use_tc_tiling_on_sc=False),
        name="item_model_sc_gather",
    )(_sc_body)
    return f(bus_i, typ_i, sub_i, price_i, bnd, bus_t, typ_t, sub_t, price_t)


def _tc_body(pair_r, par_r, bus_r, typ_r, sub_r, price_r, img_r,
             wc_r, bc_r, wd_r, bd_r, out_r):
    p = par_r[...]
    item = pair_r[:, 0:EMB] * (1.0 - p) + pair_r[:, EMB:2 * EMB] * p
    attrs = jnp.concatenate([bus_r[...], typ_r[...], sub_r[...]], axis=1)
    u = jnp.dot(attrs, wc_r[...], preferred_element_type=jnp.float32) + bc_r[...]
    cross = attrs * u + attrs
    img = jnp.dot(img_r[...], wd_r[...], preferred_element_type=jnp.float32)
    img = jnp.maximum(img + bd_r[...], 0.0)
    out_r[...] = jnp.concatenate([item, cross, price_r[...], img], axis=1)


def _tc_combine(pair_r, par, bus_r, typ_r, sub_r, price_r, img,
                cross_W, cross_b, dense_W, dense_b):
    blk = 1024
    grid = B // blk
    rows = pl.BlockSpec((blk, EMB), lambda i: (i, 0))
    return pl.pallas_call(
        _tc_body,
        grid=(grid,),
        in_specs=[
            pl.BlockSpec((blk, 2 * EMB), lambda i: (i, 0)),
            pl.BlockSpec((blk, 1), lambda i: (i, 0)),
            rows, rows, rows, rows,
            pl.BlockSpec((blk, 12), lambda i: (i, 0)),
            pl.BlockSpec((3 * EMB, 3 * EMB), lambda i: (0, 0)),
            pl.BlockSpec((1, 3 * EMB), lambda i: (0, 0)),
            pl.BlockSpec((12, 12), lambda i: (0, 0)),
            pl.BlockSpec((1, 12), lambda i: (0, 0)),
        ],
        out_specs=pl.BlockSpec((blk, 332), lambda i: (i, 0)),
        out_shape=jax.ShapeDtypeStruct((B, 332), jnp.float32),
    )(pair_r, par, bus_r, typ_r, sub_r, price_r, img,
      cross_W, cross_b, dense_W, dense_b)


def kernel(last_product_id, last_product_business_desc, last_product_type_desc,
           last_product_sub_category, last_product_list_price,
           last_image_embedding_pca, item_table, business_table, type_table,
           subcat_table, price_table, price_boundaries, cross_W, cross_b,
           dense_W, dense_b):
    right = last_product_id >= PAIR_SPLIT
    item_i = jnp.where(right, last_product_id - PAIR_H,
                       last_product_id).reshape(IR, CH)
    bus_i = last_product_business_desc.reshape(IR, CH)
    typ_i = last_product_type_desc.reshape(IR, CH)
    sub_i = last_product_sub_category.reshape(IR, CH)
    price_i = last_product_list_price.reshape(IR, CH)
    bnd = jnp.concatenate(
        [price_boundaries,
         jnp.full((NBND - price_boundaries.shape[0],), jnp.inf, jnp.float32)])
    item_tt = item_table.T
    pair_table = _tc_transpose(item_tt)
    par = right.astype(jnp.float32).reshape(B, 1)
    pair_r = _sc_item_gather(item_i, pair_table)
    bus_r, typ_r, sub_r, price_r = _sc_gather(
        bus_i, typ_i, sub_i, price_i, bnd,
        business_table, type_table, subcat_table, price_table)
    return _tc_combine(pair_r, par, bus_r, typ_r, sub_r, price_r,
                       last_image_embedding_pca, cross_W,
                       cross_b.reshape(1, 3 * EMB), dense_W,
                       dense_b.reshape(1, 12))


# speedup vs baseline: 4.7549x; 1.0023x over previous
import functools

import jax
import jax.numpy as jnp
from jax import lax
from jax.experimental import pallas as pl
from jax.experimental.pallas import tpu as pltpu
from jax.experimental.pallas import tpu_sc as plsc

B = 16384
EMB = 64
ITEM_V = 1000000
PAIR_H = 491520
PAIR_R = 508480
PAIR_SPLIT = PAIR_R
TBLK = 16384
NC = 2
NS = 16
NW = NC * NS
BPW = B // NW
CH = 128
NCH = BPW // CH
IR = B // CH
NBND = 1024


def _tc_transpose_body(l_ref, r_ref, out_ref):
    out_ref[...] = jnp.concatenate([l_ref[...].T, r_ref[...].T], axis=1)


def _tc_transpose(item_tt):
    grid = (PAIR_R + TBLK - 1) // TBLK
    return pl.pallas_call(
        _tc_transpose_body,
        grid=(grid,),
        in_specs=[
            pl.BlockSpec((EMB, TBLK), lambda i: (0, i)),
            pl.BlockSpec((EMB, TBLK), lambda i: (0, PAIR_H // TBLK + i)),
        ],
        out_specs=pl.BlockSpec((TBLK, 2 * EMB), lambda i: (i, 0)),
        out_shape=jax.ShapeDtypeStruct((PAIR_R, 2 * EMB), jnp.float32),
    )(item_tt, item_tt)


def _sc_item_body(item_i, item_t, item_o, idx_v, pair_a, pair_b, sem_a, sem_b):
    wid = lax.axis_index("s") * NC + lax.axis_index("c")
    rbase = wid * NCH
    obase = wid * BPW
    pltpu.sync_copy(item_i.at[pl.ds(rbase, NCH)], idx_v)
    bufs = [pair_a, pair_b]
    sems = [sem_a, sem_b]
    copies = [None, None]

    def fire(j):
        copies[j % 2] = pltpu.async_copy(
            item_t.at[idx_v.at[j]], bufs[j % 2], sems[j % 2])

    fire(0)
    fire(1)
    for j in range(NCH):
        copies[j % 2].wait()
        pltpu.sync_copy(bufs[j % 2], item_o.at[pl.ds(obase + j * CH, CH)])
        if j + 2 < NCH:
            fire(j + 2)


def _sc_body(bus_i, typ_i, sub_i, price_h, bnd_h,
             bus_t, typ_t, sub_t, price_t,
             bus_o, typ_o, sub_o, price_o,
             idx_v, bins_v, price_v, bnd_v, tab_v, out_v, sem):
    wid = lax.axis_index("s") * NC + lax.axis_index("c")
    base = wid * BPW

    pltpu.sync_copy(bus_i.at[pl.ds(base, BPW)], idx_v.at[pl.ds(0, BPW)])
    pltpu.sync_copy(typ_i.at[pl.ds(base, BPW)], idx_v.at[pl.ds(BPW, BPW)])
    pltpu.sync_copy(sub_i.at[pl.ds(base, BPW)], idx_v.at[pl.ds(2 * BPW, BPW)])
    pltpu.sync_copy(price_h.at[pl.ds(base, BPW)], price_v)
    pltpu.sync_copy(bnd_h, bnd_v)

    for g in range(BPW // 16):
        v = price_v[pl.ds(g * 16, 16)]
        pos = jnp.zeros((16,), jnp.int32)
        n = NBND
        while n > 1:
            half = n // 2
            probe = plsc.load_gather(bnd_v, [pos + (half - 1)])
            pos = pos + jnp.where(probe < v, half, 0)
            n -= half
        probe = plsc.load_gather(bnd_v, [pos])
        pos = pos + jnp.where(probe < v, 1, 0)
        bins_v[pl.ds(g * 16, 16)] = pos

    for t, tab in enumerate([bus_t, typ_t, sub_t, price_t]):
        pltpu.sync_copy(tab, tab_v)

        def group(g, _):
            if t < 3:
                cols = idx_v[pl.ds(t * BPW + g * 16, 16)]
            else:
                cols = bins_v[pl.ds(g * 16, 16)]
            for d in range(EMB):
                v = plsc.load_gather(tab_v, [jnp.full((16,), d, jnp.int32),
                                             cols])
                out_v[d, pl.ds(g * 16, 16)] = v
            return 0

        lax.fori_loop(0, BPW // 16, group, 0)
        out = [bus_o, typ_o, sub_o, price_o][t]
        pltpu.sync_copy(out_v, out.at[:, pl.ds(base, BPW)])


def _sc_item_gather(item_i, item_t):
    f = functools.partial(
        pl.kernel,
        out_type=jax.ShapeDtypeStruct((B, 2 * EMB), jnp.float32),
        mesh=plsc.VectorSubcoreMesh(core_axis_name="c", subcore_axis_name="s"),
        scratch_types=[
            pltpu.VMEM((NCH, CH), jnp.int32),
            pltpu.VMEM((CH, 2 * EMB), jnp.float32),
            pltpu.VMEM((CH, 2 * EMB), jnp.float32),
            pltpu.SemaphoreType.DMA,
            pltpu.SemaphoreType.DMA,
        ],
        compiler_params=pltpu.CompilerParams(needs_layout_passes=False,
                                             use_tc_tiling_on_sc=True),
        name="item_model_sc_item_gather",
    )(_sc_item_body)
    return f(item_i, item_t)


def _sc_gather(bus_i, typ_i, sub_i, price_i, bnd,
               bus_tt, typ_tt, sub_tt, price_tt):
    col = jax.ShapeDtypeStruct((EMB, B), jnp.float32)
    f = functools.partial(
        pl.kernel,
        out_type=[col] * 4,
        mesh=plsc.VectorSubcoreMesh(core_axis_name="c", subcore_axis_name="s"),
        scratch_types=[
            pltpu.VMEM((3 * BPW,), jnp.int32),
            pltpu.VMEM((BPW,), jnp.int32),
            pltpu.VMEM((BPW,), jnp.float32),
            pltpu.VMEM((NBND,), jnp.float32),
            pltpu.VMEM((EMB, 1001), jnp.float32),
            pltpu.VMEM((EMB, BPW), jnp.float32),
            pltpu.SemaphoreType.DMA,
        ],
        compiler_params=pltpu.CompilerParams(needs_layout_passes=False,
                                             use_tc_tiling_on_sc=True),
        name="item_model_sc_gather",
    )(_sc_body)
    return f(bus_i, typ_i, sub_i, price_i, bnd, bus_tt, typ_tt, sub_tt,
             price_tt)


def _tc_body(pair_r, par_r, bus_r, typ_r, sub_r, price_r, img_r,
             wc_r, bc_r, wd_r, bd_r, out_r):
    p = par_r[...]
    pair_t = pair_r[...].T
    item = pair_t[0:EMB, :] * (1.0 - p) + pair_t[EMB:2 * EMB, :] * p
    attrs = jnp.concatenate([bus_r[...], typ_r[...], sub_r[...]], axis=0)
    u = jax.lax.dot_general(wc_r[...], attrs, (((0,), (0,)), ((), ())),
                            preferred_element_type=jnp.float32) + bc_r[...]
    cross = attrs * u + attrs
    img = jax.lax.dot_general(wd_r[...], img_r[...], (((0,), (0,)), ((), ())),
                              preferred_element_type=jnp.float32)
    img = jnp.maximum(img + bd_r[...], 0.0)
    out_r[...] = jnp.concatenate([item, cross, price_r[...], img], axis=0)


def _tc_combine(pair_r, par, bus_c, typ_c, sub_c, price_c, img_t,
                cross_W, cross_b, dense_W, dense_b):
    blk = 1024
    grid = B // blk
    cols = pl.BlockSpec((EMB, blk), lambda i: (0, i))
    out_t = pl.pallas_call(
        _tc_body,
        grid=(grid,),
        in_specs=[
            pl.BlockSpec((blk, 2 * EMB), lambda i: (i, 0)),
            pl.BlockSpec((1, blk), lambda i: (0, i)),
            cols, cols, cols, cols,
            pl.BlockSpec((12, blk), lambda i: (0, i)),
            pl.BlockSpec((3 * EMB, 3 * EMB), lambda i: (0, 0)),
            pl.BlockSpec((3 * EMB, 1), lambda i: (0, 0)),
            pl.BlockSpec((12, 12), lambda i: (0, 0)),
            pl.BlockSpec((12, 1), lambda i: (0, 0)),
        ],
        out_specs=pl.BlockSpec((332, blk), lambda i: (0, i)),
        out_shape=jax.ShapeDtypeStruct((332, B), jnp.float32),
    )(pair_r, par, bus_c, typ_c, sub_c, price_c, img_t,
      cross_W, cross_b, dense_W, dense_b)
    return out_t.T


def kernel(last_product_id, last_product_business_desc, last_product_type_desc,
           last_product_sub_category, last_product_list_price,
           last_image_embedding_pca, item_table, business_table, type_table,
           subcat_table, price_table, price_boundaries, cross_W, cross_b,
           dense_W, dense_b):
    right = last_product_id >= PAIR_SPLIT
    item_i = jnp.where(right, last_product_id - PAIR_H,
                       last_product_id).reshape(IR, CH)
    bnd = jnp.concatenate(
        [price_boundaries,
         jnp.full((NBND - price_boundaries.shape[0],), jnp.inf, jnp.float32)])
    item_tt = item_table.T
    pair_table = _tc_transpose(item_tt)
    par = right.astype(jnp.float32).reshape(1, B)
    pair_r = _sc_item_gather(item_i, pair_table)
    bus_c, typ_c, sub_c, price_c = _sc_gather(
        last_product_business_desc, last_product_type_desc,
        last_product_sub_category, last_product_list_price, bnd,
        business_table.T, type_table.T, subcat_table.T, price_table.T)
    return _tc_combine(pair_r, par, bus_c, typ_c, sub_c, price_c,
                       last_image_embedding_pca.T, cross_W,
                       cross_b.reshape(3 * EMB, 1), dense_W,
                       dense_b.reshape(12, 1))


# speedup vs baseline: 4.7789x; 1.0051x over previous
import functools

import jax
import jax.numpy as jnp
from jax import lax
from jax.experimental import pallas as pl
from jax.experimental.pallas import tpu as pltpu
from jax.experimental.pallas import tpu_sc as plsc

B = 16384
EMB = 64
ITEM_V = 1000000
PAIR_H = 491520
PAIR_R = 508480
PAIR_SPLIT = PAIR_R
TBLK = 16384
NC = 2
NS = 16
NW = NC * NS
BPW = B // NW
CH = 128
NCH = BPW // CH
IR = B // CH
NBND = 1024


def _tc_transpose_body(l_ref, r_ref, out_ref):
    out_ref[...] = jnp.concatenate([l_ref[...].T, r_ref[...].T], axis=1)


def _tc_transpose(item_tt):
    grid = (PAIR_R + TBLK - 1) // TBLK
    return pl.pallas_call(
        _tc_transpose_body,
        grid=(grid,),
        in_specs=[
            pl.BlockSpec((EMB, TBLK), lambda i: (0, i)),
            pl.BlockSpec((EMB, TBLK), lambda i: (0, PAIR_H // TBLK + i)),
        ],
        out_specs=pl.BlockSpec((TBLK, 2 * EMB), lambda i: (i, 0)),
        out_shape=jax.ShapeDtypeStruct((PAIR_R, 2 * EMB), jnp.float32),
    )(item_tt, item_tt)


def _sc_item_body(item_i, item_t, item_o, idx_v, pair_a, pair_b, sem_a, sem_b):
    wid = lax.axis_index("s") * NC + lax.axis_index("c")
    rbase = wid * NCH
    obase = wid * BPW
    pltpu.sync_copy(item_i.at[pl.ds(rbase, NCH)], idx_v)
    bufs = [pair_a, pair_b]
    sems = [sem_a, sem_b]
    copies = [None, None]

    def fire(j):
        copies[j % 2] = pltpu.async_copy(
            item_t.at[idx_v.at[j]], bufs[j % 2], sems[j % 2])

    fire(0)
    fire(1)
    for j in range(NCH):
        copies[j % 2].wait()
        pltpu.sync_copy(bufs[j % 2], item_o.at[pl.ds(obase + j * CH, CH)])
        if j + 2 < NCH:
            fire(j + 2)


def _sc_body(bus_i, typ_i, sub_i, price_h, bnd_h,
             bus_t, typ_t, sub_t, price_t,
             bus_o, typ_o, sub_o, price_o,
             idx_v, bins_v, price_v, bnd_v, tab_v, out_v, sem):
    wid = lax.axis_index("s") * NC + lax.axis_index("c")
    base = wid * BPW

    pltpu.sync_copy(bus_i.at[pl.ds(base, BPW)], idx_v.at[pl.ds(0, BPW)])
    pltpu.sync_copy(typ_i.at[pl.ds(base, BPW)], idx_v.at[pl.ds(BPW, BPW)])
    pltpu.sync_copy(sub_i.at[pl.ds(base, BPW)], idx_v.at[pl.ds(2 * BPW, BPW)])
    pltpu.sync_copy(price_h.at[pl.ds(base, BPW)], price_v)
    pltpu.sync_copy(bnd_h, bnd_v)

    for g in range(BPW // 16):
        v = price_v[pl.ds(g * 16, 16)]
        pos = jnp.zeros((16,), jnp.int32)
        n = NBND
        while n > 1:
            half = n // 2
            probe = plsc.load_gather(bnd_v, [pos + (half - 1)])
            pos = pos + jnp.where(probe < v, half, 0)
            n -= half
        probe = plsc.load_gather(bnd_v, [pos])
        pos = pos + jnp.where(probe < v, 1, 0)
        bins_v[pl.ds(g * 16, 16)] = pos

    for t, tab in enumerate([bus_t, typ_t, sub_t, price_t]):
        pltpu.sync_copy(tab, tab_v)

        def group(g, _):
            if t < 3:
                cols = idx_v[pl.ds(t * BPW + g * 16, 16)]
            else:
                cols = bins_v[pl.ds(g * 16, 16)]
            for d in range(EMB):
                v = plsc.load_gather(tab_v, [jnp.full((16,), d, jnp.int32),
                                             cols])
                out_v[d, pl.ds(g * 16, 16)] = v
            return 0

        lax.fori_loop(0, BPW // 16, group, 0)
        out = [bus_o, typ_o, sub_o, price_o][t]
        pltpu.sync_copy(out_v, out.at[:, pl.ds(base, BPW)])


def _sc_item_gather(item_i, item_t):
    f = functools.partial(
        pl.kernel,
        out_type=jax.ShapeDtypeStruct((B, 2 * EMB), jnp.float32),
        mesh=plsc.VectorSubcoreMesh(core_axis_name="c", subcore_axis_name="s"),
        scratch_types=[
            pltpu.VMEM((NCH, CH), jnp.int32),
            pltpu.VMEM((CH, 2 * EMB), jnp.float32),
            pltpu.VMEM((CH, 2 * EMB), jnp.float32),
            pltpu.SemaphoreType.DMA,
            pltpu.SemaphoreType.DMA,
        ],
        compiler_params=pltpu.CompilerParams(needs_layout_passes=False,
                                             use_tc_tiling_on_sc=True),
        name="item_model_sc_item_gather",
    )(_sc_item_body)
    return f(item_i, item_t)


def _sc_gather(bus_i, typ_i, sub_i, price_i, bnd,
               bus_tt, typ_tt, sub_tt, price_tt):
    col = jax.ShapeDtypeStruct((EMB, B), jnp.float32)
    f = functools.partial(
        pl.kernel,
        out_type=[col] * 4,
        mesh=plsc.VectorSubcoreMesh(core_axis_name="c", subcore_axis_name="s"),
        scratch_types=[
            pltpu.VMEM((3 * BPW,), jnp.int32),
            pltpu.VMEM((BPW,), jnp.int32),
            pltpu.VMEM((BPW,), jnp.float32),
            pltpu.VMEM((NBND,), jnp.float32),
            pltpu.VMEM((EMB, 1001), jnp.float32),
            pltpu.VMEM((EMB, BPW), jnp.float32),
            pltpu.SemaphoreType.DMA,
        ],
        compiler_params=pltpu.CompilerParams(needs_layout_passes=False,
                                             use_tc_tiling_on_sc=True),
        name="item_model_sc_gather",
    )(_sc_body)
    return f(bus_i, typ_i, sub_i, price_i, bnd, bus_tt, typ_tt, sub_tt,
             price_tt)


def _tc_body(pair_r, par_r, bus_r, typ_r, sub_r, price_r, img_r,
             wc_r, bc_r, wd_r, bd_r, out_r):
    p = par_r[...]
    pair_t = pair_r[...].T
    item = pair_t[0:EMB, :] * (1.0 - p) + pair_t[EMB:2 * EMB, :] * p
    attrs = jnp.concatenate([bus_r[...], typ_r[...], sub_r[...]], axis=0)
    u = jax.lax.dot_general(wc_r[...], attrs, (((0,), (0,)), ((), ())),
                            preferred_element_type=jnp.float32) + bc_r[...]
    cross = attrs * u + attrs
    img = jax.lax.dot_general(wd_r[...], img_r[...], (((0,), (0,)), ((), ())),
                              preferred_element_type=jnp.float32)
    img = jnp.maximum(img + bd_r[...], 0.0)
    out_r[...] = jnp.concatenate([item, cross, price_r[...], img], axis=0)


def _tc_combine(pair_r, par, bus_c, typ_c, sub_c, price_c, img_t,
                cross_W, cross_b, dense_W, dense_b):
    blk = 1024
    grid = B // blk
    cols = pl.BlockSpec((EMB, blk), lambda i: (0, i))
    out_t = pl.pallas_call(
        _tc_body,
        grid=(grid,),
        in_specs=[
            pl.BlockSpec((blk, 2 * EMB), lambda i: (i, 0)),
            pl.BlockSpec((1, blk), lambda i: (0, i)),
            cols, cols, cols, cols,
            pl.BlockSpec((12, blk), lambda i: (0, i)),
            pl.BlockSpec((3 * EMB, 3 * EMB), lambda i: (0, 0)),
            pl.BlockSpec((3 * EMB, 1), lambda i: (0, 0)),
            pl.BlockSpec((12, 12), lambda i: (0, 0)),
            pl.BlockSpec((12, 1), lambda i: (0, 0)),
        ],
        out_specs=pl.BlockSpec((332, blk), lambda i: (0, i)),
        out_shape=jax.ShapeDtypeStruct((332, B), jnp.float32),
    )(pair_r, par, bus_c, typ_c, sub_c, price_c, img_t,
      cross_W, cross_b, dense_W, dense_b)
    return out_t.T


def kernel(last_product_id, last_product_business_desc, last_product_type_desc,
           last_product_sub_category, last_product_list_price,
           last_image_embedding_pca, item_table, business_table, type_table,
           subcat_table, price_table, price_boundaries, cross_W, cross_b,
           dense_W, dense_b):
    right = last_product_id >= PAIR_SPLIT
    item_i = jnp.where(right, last_product_id - PAIR_H,
                       last_product_id).reshape(IR, CH)
    bnd = jnp.concatenate(
        [price_boundaries,
         jnp.full((NBND - price_boundaries.shape[0],), jnp.inf, jnp.float32)])
    item_tt = item_table.T
    par = right.astype(jnp.float32).reshape(1, B)
    bus_c, typ_c, sub_c, price_c = _sc_gather(
        last_product_business_desc, last_product_type_desc,
        last_product_sub_category, last_product_list_price, bnd,
        business_table.T, type_table.T, subcat_table.T, price_table.T)
    pair_table = _tc_transpose(item_tt)
    pair_r = _sc_item_gather(item_i, pair_table)
    return _tc_combine(pair_r, par, bus_c, typ_c, sub_c, price_c,
                       last_image_embedding_pca.T, cross_W,
                       cross_b.reshape(3 * EMB, 1), dense_W,
                       dense_b.reshape(12, 1))


# speedup vs baseline: 5.7887x; 1.2113x over previous
import functools

import jax
import jax.numpy as jnp
from jax import lax
from jax.experimental import pallas as pl
from jax.experimental.pallas import tpu as pltpu
from jax.experimental.pallas import tpu_sc as plsc

B = 16384
EMB = 64
ITEM_V = 1000000
PAIR_H = 491520
PAIR_R = 508480
PAIR_SPLIT = PAIR_R
TBLK = 16384
NC = 2
NS = 16
NW = NC * NS
BPW = B // NW
CH = 128
NCH = BPW // CH
IR = B // CH
NBND = 1024


def _tc_transpose_body(l_ref, r_ref, out_ref):
    out_ref[...] = jnp.concatenate([l_ref[...].T, r_ref[...].T], axis=1)


def _tc_transpose(item_tt):
    grid = (PAIR_R + TBLK - 1) // TBLK
    return pl.pallas_call(
        _tc_transpose_body,
        grid=(grid,),
        in_specs=[
            pl.BlockSpec((EMB, TBLK), lambda i: (0, i)),
            pl.BlockSpec((EMB, TBLK), lambda i: (0, PAIR_H // TBLK + i)),
        ],
        out_specs=pl.BlockSpec((TBLK, 2 * EMB), lambda i: (i, 0)),
        out_shape=jax.ShapeDtypeStruct((PAIR_R, 2 * EMB), jnp.float32),
    )(item_tt, item_tt)


def _sc_item_body(item_i, item_t, item_o, idx_v, pair_a, pair_b, sem_a, sem_b):
    wid = lax.axis_index("s") * NC + lax.axis_index("c")
    rbase = wid * NCH
    obase = wid * BPW
    pltpu.sync_copy(item_i.at[pl.ds(rbase, NCH)], idx_v)
    bufs = [pair_a, pair_b]
    sems = [sem_a, sem_b]
    copies = [None, None]

    def fire(j):
        copies[j % 2] = pltpu.async_copy(
            item_t.at[idx_v.at[j]], bufs[j % 2], sems[j % 2])

    fire(0)
    fire(1)
    for j in range(NCH):
        copies[j % 2].wait()
        pltpu.sync_copy(bufs[j % 2], item_o.at[pl.ds(obase + j * CH, CH)])
        if j + 2 < NCH:
            fire(j + 2)


def _sc_body(bus_i, typ_i, sub_i, price_h, bnd_h,
             bus_t, typ_t, sub_t, price_t,
             bus_o, typ_o, sub_o, price_o,
             idx_v, bins_v, price_v, bnd_v, tab_v, out_v, sem):
    wid = lax.axis_index("s") * NC + lax.axis_index("c")
    base = wid * BPW

    pltpu.sync_copy(bus_i.at[pl.ds(base, BPW)], idx_v.at[pl.ds(0, BPW)])
    pltpu.sync_copy(typ_i.at[pl.ds(base, BPW)], idx_v.at[pl.ds(BPW, BPW)])
    pltpu.sync_copy(sub_i.at[pl.ds(base, BPW)], idx_v.at[pl.ds(2 * BPW, BPW)])
    pltpu.sync_copy(price_h.at[pl.ds(base, BPW)], price_v)
    pltpu.sync_copy(bnd_h, bnd_v)

    for g in range(BPW // 16):
        v = price_v[pl.ds(g * 16, 16)]
        pos = jnp.zeros((16,), jnp.int32)
        n = NBND
        while n > 1:
            half = n // 2
            probe = plsc.load_gather(bnd_v, [pos + (half - 1)])
            pos = pos + jnp.where(probe < v, half, 0)
            n -= half
        probe = plsc.load_gather(bnd_v, [pos])
        pos = pos + jnp.where(probe < v, 1, 0)
        bins_v[pl.ds(g * 16, 16)] = pos

    for t, tab in enumerate([bus_t, typ_t, sub_t, price_t]):
        pltpu.sync_copy(tab, tab_v)

        def group(g, _):
            if t < 3:
                cols = idx_v[pl.ds(t * BPW + g * 16, 16)]
            else:
                cols = bins_v[pl.ds(g * 16, 16)]
            for d in range(EMB):
                v = plsc.load_gather(tab_v, [jnp.full((16,), d, jnp.int32),
                                             cols])
                out_v[d, pl.ds(g * 16, 16)] = v
            return 0

        lax.fori_loop(0, BPW // 16, group, 0)
        out = [bus_o, typ_o, sub_o, price_o][t]
        pltpu.sync_copy(out_v, out.at[:, pl.ds(base, BPW)])


def _sc_item_gather(item_i, item_t):
    f = functools.partial(
        pl.kernel,
        out_type=jax.ShapeDtypeStruct((B, 2 * EMB), jnp.float32),
        mesh=plsc.VectorSubcoreMesh(core_axis_name="c", subcore_axis_name="s"),
        scratch_types=[
            pltpu.VMEM((NCH, CH), jnp.int32),
            pltpu.VMEM((CH, 2 * EMB), jnp.float32),
            pltpu.VMEM((CH, 2 * EMB), jnp.float32),
            pltpu.SemaphoreType.DMA,
            pltpu.SemaphoreType.DMA,
        ],
        compiler_params=pltpu.CompilerParams(needs_layout_passes=False,
                                             use_tc_tiling_on_sc=True),
        name="item_model_sc_item_gather",
    )(_sc_item_body)
    return f(item_i, item_t)


def _sc_gather(bus_i, typ_i, sub_i, price_i, bnd,
               bus_tt, typ_tt, sub_tt, price_tt):
    col = jax.ShapeDtypeStruct((EMB, B), jnp.float32)
    f = functools.partial(
        pl.kernel,
        out_type=[col] * 4,
        mesh=plsc.VectorSubcoreMesh(core_axis_name="c", subcore_axis_name="s"),
        scratch_types=[
            pltpu.VMEM((3 * BPW,), jnp.int32),
            pltpu.VMEM((BPW,), jnp.int32),
            pltpu.VMEM((BPW,), jnp.float32),
            pltpu.VMEM((NBND,), jnp.float32),
            pltpu.VMEM((EMB, 1001), jnp.float32),
            pltpu.VMEM((EMB, BPW), jnp.float32),
            pltpu.SemaphoreType.DMA,
        ],
        compiler_params=pltpu.CompilerParams(needs_layout_passes=False,
                                             use_tc_tiling_on_sc=True),
        name="item_model_sc_gather",
    )(_sc_body)
    return f(bus_i, typ_i, sub_i, price_i, bnd, bus_tt, typ_tt, sub_tt,
             price_tt)


def _tc_body(pair_r, par_r, bus_r, typ_r, sub_r, price_r, img_r,
             wc_r, bc_r, wd_r, bd_r, out_r):
    p = par_r[...]
    pair_t = pair_r[...].T
    item = pair_t[0:EMB, :] * (1.0 - p) + pair_t[EMB:2 * EMB, :] * p
    attrs = jnp.concatenate([bus_r[...], typ_r[...], sub_r[...]], axis=0)
    u = jax.lax.dot_general(wc_r[...], attrs, (((0,), (0,)), ((), ())),
                            preferred_element_type=jnp.float32) + bc_r[...]
    cross = attrs * u + attrs
    img = jax.lax.dot_general(wd_r[...], img_r[...], (((0,), (0,)), ((), ())),
                              preferred_element_type=jnp.float32)
    img = jnp.maximum(img + bd_r[...], 0.0)
    out_r[...] = jnp.concatenate([item, cross, price_r[...], img], axis=0)


def _tc_combine(pair_r, par, bus_c, typ_c, sub_c, price_c, img_t,
                cross_W, cross_b, dense_W, dense_b):
    blk = 1024
    grid = B // blk
    cols = pl.BlockSpec((EMB, blk), lambda i: (0, i))
    out_t = pl.pallas_call(
        _tc_body,
        grid=(grid,),
        in_specs=[
            pl.BlockSpec((blk, 2 * EMB), lambda i: (i, 0)),
            pl.BlockSpec((1, blk), lambda i: (0, i)),
            cols, cols, cols, cols,
            pl.BlockSpec((12, blk), lambda i: (0, i)),
            pl.BlockSpec((3 * EMB, 3 * EMB), lambda i: (0, 0)),
            pl.BlockSpec((3 * EMB, 1), lambda i: (0, 0)),
            pl.BlockSpec((12, 12), lambda i: (0, 0)),
            pl.BlockSpec((12, 1), lambda i: (0, 0)),
        ],
        out_specs=pl.BlockSpec((332, blk), lambda i: (0, i)),
        out_shape=jax.ShapeDtypeStruct((332, B), jnp.float32),
    )(pair_r, par, bus_c, typ_c, sub_c, price_c, img_t,
      cross_W, cross_b, dense_W, dense_b)
    return out_t.T


def kernel(last_product_id, last_product_business_desc, last_product_type_desc,
           last_product_sub_category, last_product_list_price,
           last_image_embedding_pca, item_table, business_table, type_table,
           subcat_table, price_table, price_boundaries, cross_W, cross_b,
           dense_W, dense_b):
    right = last_product_id >= PAIR_SPLIT
    item_i = jnp.where(right, last_product_id - PAIR_H,
                       last_product_id).reshape(IR, CH)
    bnd = jnp.concatenate(
        [price_boundaries,
         jnp.full((NBND - price_boundaries.shape[0],), jnp.inf, jnp.float32)])
    item_tt = item_table.T
    par = right.astype(jnp.float32).reshape(1, B)
    bus_c, typ_c, sub_c, price_c = _sc_gather(
        last_product_business_desc, last_product_type_desc,
        last_product_sub_category, last_product_list_price, bnd,
        business_table.T, type_table.T, subcat_table.T, price_table.T)
    pair_table = _tc_transpose(item_tt)
    item_i, bus_c = jax.lax.optimization_barrier((item_i, bus_c))
    pair_r = _sc_item_gather(item_i, pair_table)
    return _tc_combine(pair_r, par, bus_c, typ_c, sub_c, price_c,
                       last_image_embedding_pca.T, cross_W,
                       cross_b.reshape(3 * EMB, 1), dense_W,
                       dense_b.reshape(12, 1))
